# baseline (speedup 1.0000x reference)
import jax
import jax.numpy as jnp
from jax import lax
from jax.experimental import pallas as pl
from jax.experimental.pallas import tpu as pltpu
from jax.experimental.pallas import tpu_sc as plsc

BS = 8
H = 256
NV = 6890
NF = 13776

NC = 2
NS = 16
L = 16
W = NC * NS
WPB = W // BS
RPW = H // WPB
RCH = 8
NVUP = 6896
FCH = 1968


def _body(cams_hbm, verts_hbm, faces_hbm, fim_hbm, out_hbm,
          cams_v, verts_v, faces_v, px_v, py_v, bcx_v, bcy_v, fim_v, out_v):
    s = lax.axis_index("s")
    c = lax.axis_index("c")
    wid = s * NC + c
    b = wid // WPB
    q = wid % WPB

    pltpu.sync_copy(cams_hbm, cams_v)
    pltpu.sync_copy(verts_hbm.at[b], verts_v)

    iota = lax.iota(jnp.int32, L)
    zero = iota * 0

    bsplat = zero + b
    cam0 = plsc.load_gather(cams_v, [bsplat, zero])
    cam1 = plsc.load_gather(cams_v, [bsplat, zero + 1])
    cam2 = plsc.load_gather(cams_v, [bsplat, zero + 2])

    def p1(i, carry):
        vi = iota + i * L
        xv = plsc.load_gather(verts_v, [vi, zero])
        yv = plsc.load_gather(verts_v, [vi, zero + 1])
        px_v[pl.ds(i * L, L)] = cam0 * (xv + cam1)
        py_v[pl.ds(i * L, L)] = cam0 * (yv + cam2)
        return carry

    lax.fori_loop(0, NVUP // L, p1, 0, unroll=4)

    def p2chunk(ch, carry):
        f0g = ch * FCH
        pltpu.sync_copy(faces_hbm.at[pl.ds(f0g, FCH)], faces_v)

        def p2(i, icarry):
            fi = iota + i * L
            f0 = plsc.load_gather(faces_v, [fi, zero])
            f1 = plsc.load_gather(faces_v, [fi, zero + 1])
            f2 = plsc.load_gather(faces_v, [fi, zero + 2])
            sx = (plsc.load_gather(px_v, [f0]) + plsc.load_gather(px_v, [f1])
                  + plsc.load_gather(px_v, [f2]))
            sy = (plsc.load_gather(py_v, [f0]) + plsc.load_gather(py_v, [f1])
                  + plsc.load_gather(py_v, [f2]))
            bcx_v[pl.ds(f0g + i * L, L)] = sx * jnp.float32(1.0 / 3.0)
            bcy_v[pl.ds(f0g + i * L, L)] = sy * jnp.float32(1.0 / 3.0)
            return icarry

        lax.fori_loop(0, FCH // L, p2, 0, unroll=4)
        return carry

    lax.fori_loop(0, NF // FCH, p2chunk, 0)

    row0 = q * RPW

    def chunk(ci, carry):
        r0 = row0 + ci * RCH
        pltpu.sync_copy(fim_hbm.at[b, pl.ds(r0, RCH)], fim_v)

        def row(r, rcarry):
            rsplat = zero + r

            def col(v, ccarry):
                t = fim_v[r, pl.ds(v * L, L)]
                mask = t >= 0
                tc = jnp.minimum(jnp.maximum(t, 0), NF - 1)
                gx = plsc.load_gather(bcx_v, [tc])
                gy = plsc.load_gather(bcy_v, [tc])
                rx = jnp.where(mask, gx, jnp.float32(-1.0))
                ry = jnp.where(mask, gy, jnp.float32(-1.0))
                cidx = iota * 2 + v * (L * 2)
                plsc.store_scatter(out_v, [rsplat, cidx], rx)
                plsc.store_scatter(out_v, [rsplat, cidx + 1], ry)
                return ccarry

            return lax.fori_loop(0, H // L, col, rcarry, unroll=4)

        lax.fori_loop(0, RCH, row, 0)
        pltpu.sync_copy(out_v, out_hbm.at[b, pl.ds(r0, RCH)])
        return carry

    lax.fori_loop(0, RPW // RCH, chunk, 0)


@jax.jit
def _run(cams, verts, faces_a, fim):
    mesh = plsc.VectorSubcoreMesh(core_axis_name="c", subcore_axis_name="s",
                                  num_cores=NC, num_subcores=NS)
    f = pl.kernel(
        _body,
        out_type=jax.ShapeDtypeStruct((BS, H, 2 * H), jnp.float32),
        mesh=mesh,
        compiler_params=pltpu.CompilerParams(needs_layout_passes=False,
                                             use_tc_tiling_on_sc=False),
        scratch_types=[
            pltpu.VMEM((BS, 3), jnp.float32),
            pltpu.VMEM((NVUP, 3), jnp.float32),
            pltpu.VMEM((FCH, 3), jnp.int32),
            pltpu.VMEM((NVUP,), jnp.float32),
            pltpu.VMEM((NVUP,), jnp.float32),
            pltpu.VMEM((NF,), jnp.float32),
            pltpu.VMEM((NF,), jnp.float32),
            pltpu.VMEM((RCH, H), jnp.int32),
            pltpu.VMEM((RCH, 2 * H), jnp.float32),
        ],
    )
    return f(cams, verts, faces_a, fim)


def kernel(src_cams, src_verts, faces, src_fim, tgt_fim):
    del src_fim
    verts = jnp.pad(src_verts.astype(jnp.float32), ((0, 0), (0, NVUP - NV), (0, 0)))
    out = _run(src_cams.astype(jnp.float32), verts,
               faces.astype(jnp.int32), tgt_fim.astype(jnp.int32))
    return out.reshape(BS, H, H, 2)
